# Initial kernel scaffold; baseline (speedup 1.0000x reference)
import jax
import jax.numpy as jnp
from jax.experimental import pallas as pl


def kernel(x, edge_index, W1, b1, W2, b2):
    raise NotImplementedError("write your pallas kernel here")


# speedup vs baseline: 30.6085x; 30.6085x over previous
import functools

import jax
import jax.numpy as jnp
from jax import lax
from jax.experimental import pallas as pl
from jax.experimental.pallas import tpu as pltpu
from jax.experimental.pallas import tpu_sc as plsc

N = 10000
E = 320000
D_IN = 128
D_H = 64

NC = 2
NS = 16
NW = NC * NS

N_PAD = 10240
ROWS_PS = N_PAD // NS
CHUNK = 128
NCH = 80
E_PAD = NW * NCH * CHUNK
BLK = 512
GRID = N_PAD // BLK

_mesh = plsc.VectorSubcoreMesh(core_axis_name="c", subcore_axis_name="s")


@functools.partial(
    pl.kernel,
    out_type=jax.ShapeDtypeStruct((N_PAD,), jnp.float32),
    mesh=_mesh,
    scratch_types=[
        pltpu.VMEM((NCH, CHUNK), jnp.int32),
        pltpu.VMEM((CHUNK,), jnp.float32),
        pltpu.VMEM_SHARED((N_PAD,), jnp.float32),
    ],
)
def _sc_hist(dst_hbm, ones_hbm, zeros_hbm, hist_hbm, dst_v, ones_v, hist_sp):
    c = lax.axis_index("c")
    s = lax.axis_index("s")

    @pl.when(c == 0)
    def _():
        pltpu.sync_copy(zeros_hbm.at[pl.ds(s * ROWS_PS, ROWS_PS)],
                        hist_sp.at[pl.ds(s * ROWS_PS, ROWS_PS)])
        pltpu.sync_copy(ones_hbm, ones_v)
        plsc.subcore_barrier()
        for r in range(2):
            wid = s + r * NS
            pltpu.sync_copy(dst_hbm.at[wid], dst_v)

            def step(j, carry):
                pltpu.sync_copy(ones_v, hist_sp.at[dst_v.at[j]], add=True)
                return carry

            lax.fori_loop(0, NCH, step, 0)
        plsc.subcore_barrier()
        pltpu.sync_copy(hist_sp.at[pl.ds(s * ROWS_PS, ROWS_PS)],
                        hist_hbm.at[pl.ds(s * ROWS_PS, ROWS_PS)])


@functools.partial(
    pl.kernel,
    out_type=jax.ShapeDtypeStruct((NC * N_PAD, D_H), jnp.float32),
    mesh=_mesh,
    scratch_types=[
        pltpu.VMEM((NCH, CHUNK), jnp.int32),
        pltpu.VMEM((NCH, CHUNK), jnp.int32),
        pltpu.VMEM((2, CHUNK, D_H), jnp.float32),
        pltpu.VMEM_SHARED((N_PAD, D_H), jnp.float32),
        pltpu.SemaphoreType.DMA,
    ],
    compiler_params=pltpu.CompilerParams(use_tc_tiling_on_sc=False),
)
def _sc_segsum(y_hbm, src_hbm, dst_hbm, zeros_hbm, z_hbm,
               src_v, dst_v, gbuf, z_sp, gsem):
    c = lax.axis_index("c")
    s = lax.axis_index("s")
    wid = c * NS + s

    pltpu.sync_copy(zeros_hbm.at[pl.ds(s * ROWS_PS, ROWS_PS)],
                    z_sp.at[pl.ds(s * ROWS_PS, ROWS_PS)])
    pltpu.sync_copy(src_hbm.at[wid], src_v)
    pltpu.sync_copy(dst_hbm.at[wid], dst_v)
    plsc.subcore_barrier()

    pltpu.async_copy(y_hbm.at[src_v.at[0]], gbuf.at[0], gsem)

    def step(j, carry):
        for b in (0, 1):
            cidx = 2 * j + b
            pltpu.make_async_copy(y_hbm.at[src_v.at[cidx]], gbuf.at[b],
                                  gsem).wait()
            if b == 0:
                pltpu.async_copy(y_hbm.at[src_v.at[cidx + 1]], gbuf.at[1],
                                 gsem)
            else:
                @pl.when(j < NCH // 2 - 1)
                def _():
                    pltpu.async_copy(y_hbm.at[src_v.at[cidx + 1]], gbuf.at[0],
                                     gsem)
            pltpu.sync_copy(gbuf.at[b], z_sp.at[dst_v.at[cidx]], add=True)
        return carry

    lax.fori_loop(0, NCH // 2, step, 0)
    plsc.subcore_barrier()
    pltpu.sync_copy(z_sp.at[pl.ds(s * ROWS_PS, ROWS_PS)],
                    z_hbm.at[pl.ds(c * N_PAD + s * ROWS_PS, ROWS_PS)])


def _tc_scale_matmul_body(hist_ref, x_ref, w_ref, y_ref):
    dinv = lax.rsqrt(hist_ref[...] + 1.0)
    y_ref[...] = jnp.dot(x_ref[...], w_ref[...],
                         preferred_element_type=jnp.float32) * dinv


def _tc_mid_body(hist_ref, z_ref, y_ref, w_ref, b_ref, y2_ref):
    i = pl.program_id(0)
    dinv = lax.rsqrt(hist_ref[...] + 1.0)
    zsum = z_ref[0] + z_ref[1] + y_ref[...]
    h = jnp.maximum(dinv * zsum + b_ref[...], 0.0)
    y2 = jnp.dot(h, w_ref[...], preferred_element_type=jnp.float32) * dinv
    row = i * BLK + lax.broadcasted_iota(jnp.int32, (BLK, 1), 0)
    y2_ref[...] = jnp.where(row < N, y2, 0.0)


def _tc_final_body(hist_ref, z_ref, y_ref, b_ref, out_ref):
    dinv = lax.rsqrt(hist_ref[...] + 1.0)
    zsum = z_ref[0] + z_ref[1] + y_ref[...]
    out_ref[...] = jnp.maximum(dinv * zsum + b_ref[...], 0.0)


_hist_spec = pl.BlockSpec((BLK, 1), lambda i: (i, 0))
_row_spec = pl.BlockSpec((BLK, D_H), lambda i: (i, 0))
_z_spec = pl.BlockSpec((2, BLK, D_H), lambda i: (0, i, 0))
_b_spec = pl.BlockSpec((1, D_H), lambda i: (0, 0))

_tc_scale_matmul = pl.pallas_call(
    _tc_scale_matmul_body,
    grid=(GRID,),
    in_specs=[_hist_spec,
              pl.BlockSpec((BLK, D_IN), lambda i: (i, 0)),
              pl.BlockSpec((D_IN, D_H), lambda i: (0, 0))],
    out_specs=_row_spec,
    out_shape=jax.ShapeDtypeStruct((N_PAD, D_H), jnp.float32),
)

_tc_mid = pl.pallas_call(
    _tc_mid_body,
    grid=(GRID,),
    in_specs=[_hist_spec, _z_spec, _row_spec,
              pl.BlockSpec((D_H, D_H), lambda i: (0, 0)), _b_spec],
    out_specs=_row_spec,
    out_shape=jax.ShapeDtypeStruct((N_PAD, D_H), jnp.float32),
)

_tc_final = pl.pallas_call(
    _tc_final_body,
    grid=(GRID,),
    in_specs=[_hist_spec, _z_spec, _row_spec, _b_spec],
    out_specs=_row_spec,
    out_shape=jax.ShapeDtypeStruct((N_PAD, D_H), jnp.float32),
)


def kernel(x, edge_index, W1, b1, W2, b2):
    src = edge_index[0]
    dst = edge_index[1]

    npad_e = E_PAD - E
    pad_idx = (N + (jnp.arange(npad_e, dtype=jnp.int32) % (N_PAD - N)))
    src_p = jnp.concatenate([src, pad_idx]).reshape(NW, NCH, CHUNK)
    dst_p = jnp.concatenate([dst, pad_idx]).reshape(NW, NCH, CHUNK)

    x_pad = jnp.zeros((N_PAD, D_IN), jnp.float32).at[:N].set(x)
    zeros1 = jnp.zeros((N_PAD,), jnp.float32)
    zeros2 = jnp.zeros((N_PAD, D_H), jnp.float32)
    ones_c = jnp.ones((CHUNK,), jnp.float32)

    hist = _sc_hist(dst_p, ones_c, zeros1)
    hist_col = hist.reshape(N_PAD, 1)

    y1 = _tc_scale_matmul(hist_col, x_pad, W1)
    z1 = _sc_segsum(y1, src_p, dst_p, zeros2).reshape(NC, N_PAD, D_H)
    y2 = _tc_mid(hist_col, z1, y1, W2, b1.reshape(1, D_H))
    z2 = _sc_segsum(y2, src_p, dst_p, zeros2).reshape(NC, N_PAD, D_H)
    out = _tc_final(hist_col, z2, y2, b2.reshape(1, D_H))
    return out[:N]


# speedup vs baseline: 39.9784x; 1.3061x over previous
import functools

import jax
import jax.numpy as jnp
from jax import lax
from jax.experimental import pallas as pl
from jax.experimental.pallas import tpu as pltpu
from jax.experimental.pallas import tpu_sc as plsc

N = 10000
E = 320000
D_IN = 128
D_H = 64

NC = 2
NS = 16
NW = NC * NS

N_PAD = 10240
ROWS_PS = N_PAD // NS
CHUNK = 128
NCH = 80
E_PAD = NW * NCH * CHUNK
BLK = 512
GRID = N_PAD // BLK

NB = 8
PF = 6

_mesh = plsc.VectorSubcoreMesh(core_axis_name="c", subcore_axis_name="s")


@functools.partial(
    pl.kernel,
    out_type=jax.ShapeDtypeStruct((NC * N_PAD,), jnp.float32),
    mesh=_mesh,
    scratch_types=[
        pltpu.VMEM((NCH, CHUNK), jnp.int32),
        pltpu.VMEM((CHUNK,), jnp.float32),
        pltpu.VMEM_SHARED((N_PAD,), jnp.float32),
        pltpu.SemaphoreType.DMA,
    ],
)
def _sc_hist(dst_hbm, ones_hbm, zeros_hbm, hist_hbm, dst_v, ones_v, hist_sp,
             ssem):
    c = lax.axis_index("c")
    s = lax.axis_index("s")
    wid = c * NS + s

    pltpu.sync_copy(zeros_hbm.at[pl.ds(s * ROWS_PS, ROWS_PS)],
                    hist_sp.at[pl.ds(s * ROWS_PS, ROWS_PS)])
    pltpu.sync_copy(ones_hbm, ones_v)
    pltpu.sync_copy(dst_hbm.at[wid], dst_v)
    plsc.subcore_barrier()

    for j0 in range(NB):
        pltpu.async_copy(ones_v, hist_sp.at[dst_v.at[j0]], ssem, add=True)

    def step(j, carry):
        pltpu.make_async_copy(ones_v, hist_sp.at[dst_v.at[j]], ssem).wait()
        pltpu.async_copy(ones_v, hist_sp.at[dst_v.at[j + NB]], ssem, add=True)
        return carry

    lax.fori_loop(0, NCH - NB, step, 0)
    for j0 in range(NB):
        pltpu.make_async_copy(ones_v, hist_sp.at[dst_v.at[j0]], ssem).wait()

    plsc.subcore_barrier()
    pltpu.sync_copy(hist_sp.at[pl.ds(s * ROWS_PS, ROWS_PS)],
                    hist_hbm.at[pl.ds(c * N_PAD + s * ROWS_PS, ROWS_PS)])


@functools.partial(
    pl.kernel,
    out_type=jax.ShapeDtypeStruct((NC * N_PAD, D_H), jnp.float32),
    mesh=_mesh,
    scratch_types=[
        pltpu.VMEM((NCH, CHUNK), jnp.int32),
        pltpu.VMEM((NCH, CHUNK), jnp.int32),
        pltpu.VMEM((NB, CHUNK, D_H), jnp.float32),
        pltpu.VMEM_SHARED((N_PAD, D_H), jnp.float32),
        pltpu.SemaphoreType.DMA,
        pltpu.SemaphoreType.DMA,
    ],
    compiler_params=pltpu.CompilerParams(use_tc_tiling_on_sc=False),
)
def _sc_segsum(y_hbm, src_hbm, dst_hbm, zeros_hbm, z_hbm,
               src_v, dst_v, gbuf, z_sp, gsem, ssem):
    c = lax.axis_index("c")
    s = lax.axis_index("s")
    wid = c * NS + s

    pltpu.sync_copy(zeros_hbm.at[pl.ds(s * ROWS_PS, ROWS_PS)],
                    z_sp.at[pl.ds(s * ROWS_PS, ROWS_PS)])
    pltpu.sync_copy(src_hbm.at[wid], src_v)
    pltpu.sync_copy(dst_hbm.at[wid], dst_v)
    plsc.subcore_barrier()

    for b0 in range(PF):
        pltpu.async_copy(y_hbm.at[src_v.at[b0]], gbuf.at[b0], gsem)

    def step(j, carry):
        for b in range(NB):
            cx = NB * j + b
            pltpu.make_async_copy(y_hbm.at[src_v.at[cx]], gbuf.at[b],
                                  gsem).wait()
            pltpu.async_copy(gbuf.at[b], z_sp.at[dst_v.at[cx]], ssem,
                             add=True)
            @pl.when(cx >= 2)
            def _():
                pltpu.make_async_copy(gbuf.at[b], z_sp.at[dst_v.at[cx]],
                                      ssem).wait()
            @pl.when(cx + PF < NCH)
            def _():
                pltpu.async_copy(y_hbm.at[src_v.at[cx + PF]],
                                 gbuf.at[(b + PF) % NB], gsem)
        return carry

    lax.fori_loop(0, NCH // NB, step, 0)
    pltpu.make_async_copy(gbuf.at[0], z_sp.at[dst_v.at[0]], ssem).wait()
    pltpu.make_async_copy(gbuf.at[0], z_sp.at[dst_v.at[0]], ssem).wait()

    plsc.subcore_barrier()
    pltpu.sync_copy(z_sp.at[pl.ds(s * ROWS_PS, ROWS_PS)],
                    z_hbm.at[pl.ds(c * N_PAD + s * ROWS_PS, ROWS_PS)])


def _tc_scale_matmul_body(hist_ref, x_ref, w_ref, y_ref, dinv_ref):
    dinv = lax.rsqrt(hist_ref[0] + hist_ref[1] + 1.0)
    dinv_ref[...] = dinv
    y_ref[...] = jnp.dot(x_ref[...], w_ref[...],
                         preferred_element_type=jnp.float32) * dinv


def _tc_mid_body(dinv_ref, z_ref, y_ref, w_ref, b_ref, y2_ref):
    i = pl.program_id(0)
    dinv = dinv_ref[...]
    zsum = z_ref[0] + z_ref[1] + y_ref[...]
    h = jnp.maximum(dinv * zsum + b_ref[...], 0.0)
    y2 = jnp.dot(h, w_ref[...], preferred_element_type=jnp.float32) * dinv
    row = i * BLK + lax.broadcasted_iota(jnp.int32, (BLK, 1), 0)
    y2_ref[...] = jnp.where(row < N, y2, 0.0)


def _tc_final_body(dinv_ref, z_ref, y_ref, b_ref, out_ref):
    zsum = z_ref[0] + z_ref[1] + y_ref[...]
    out_ref[...] = jnp.maximum(dinv_ref[...] * zsum + b_ref[...], 0.0)


_dinv_spec = pl.BlockSpec((BLK, 1), lambda i: (i, 0))
_row_spec = pl.BlockSpec((BLK, D_H), lambda i: (i, 0))
_z_spec = pl.BlockSpec((2, BLK, D_H), lambda i: (0, i, 0))
_b_spec = pl.BlockSpec((1, D_H), lambda i: (0, 0))

_tc_scale_matmul = pl.pallas_call(
    _tc_scale_matmul_body,
    grid=(GRID,),
    in_specs=[pl.BlockSpec((2, BLK, 1), lambda i: (0, i, 0)),
              pl.BlockSpec((BLK, D_IN), lambda i: (i, 0)),
              pl.BlockSpec((D_IN, D_H), lambda i: (0, 0))],
    out_specs=[_row_spec, _dinv_spec],
    out_shape=[jax.ShapeDtypeStruct((N_PAD, D_H), jnp.float32),
               jax.ShapeDtypeStruct((N_PAD, 1), jnp.float32)],
)

_tc_mid = pl.pallas_call(
    _tc_mid_body,
    grid=(GRID,),
    in_specs=[_dinv_spec, _z_spec, _row_spec,
              pl.BlockSpec((D_H, D_H), lambda i: (0, 0)), _b_spec],
    out_specs=_row_spec,
    out_shape=jax.ShapeDtypeStruct((N_PAD, D_H), jnp.float32),
)

_tc_final = pl.pallas_call(
    _tc_final_body,
    grid=(GRID,),
    in_specs=[_dinv_spec, _z_spec, _row_spec, _b_spec],
    out_specs=_row_spec,
    out_shape=jax.ShapeDtypeStruct((N_PAD, D_H), jnp.float32),
)


def kernel(x, edge_index, W1, b1, W2, b2):
    src = edge_index[0]
    dst = edge_index[1]

    npad_e = E_PAD - E
    pad_idx = (N + (jnp.arange(npad_e, dtype=jnp.int32) % (N_PAD - N)))
    src_p = jnp.concatenate([src, pad_idx]).reshape(NW, NCH, CHUNK)
    dst_p = jnp.concatenate([dst, pad_idx]).reshape(NW, NCH, CHUNK)

    x_pad = jnp.zeros((N_PAD, D_IN), jnp.float32).at[:N].set(x)
    zeros1 = jnp.zeros((N_PAD,), jnp.float32)
    zeros2 = jnp.zeros((N_PAD, D_H), jnp.float32)
    ones_c = jnp.ones((CHUNK,), jnp.float32)

    hist = _sc_hist(dst_p, ones_c, zeros1).reshape(NC, N_PAD, 1)

    y1, dinv = _tc_scale_matmul(hist, x_pad, W1)
    z1 = _sc_segsum(y1, src_p, dst_p, zeros2).reshape(NC, N_PAD, D_H)
    y2 = _tc_mid(dinv, z1, y1, W2, b1.reshape(1, D_H))
    z2 = _sc_segsum(y2, src_p, dst_p, zeros2).reshape(NC, N_PAD, D_H)
    out = _tc_final(dinv, z2, y2, b2.reshape(1, D_H))
    return out[:N]


# speedup vs baseline: 48.2100x; 1.2059x over previous
import functools

import jax
import jax.numpy as jnp
from jax import lax
from jax.experimental import pallas as pl
from jax.experimental.pallas import tpu as pltpu
from jax.experimental.pallas import tpu_sc as plsc

N = 10000
E = 320000
D_IN = 128
D_H = 64

NC = 2
NS = 16
NW = NC * NS

N_PAD = 10240
ROWS_PS = N_PAD // NS
CHUNK = 128
GRP = 1
NG = 80
E_PAD = NW * NG * GRP * CHUNK
BLK = 2048
GRID = N_PAD // BLK

NB = 8
PF = 6

_mesh = plsc.VectorSubcoreMesh(core_axis_name="c", subcore_axis_name="s")


@functools.partial(
    pl.kernel,
    out_type=jax.ShapeDtypeStruct((NC * N_PAD,), jnp.float32),
    mesh=_mesh,
    scratch_types=[
        pltpu.VMEM((NG, CHUNK), jnp.int32),
        pltpu.VMEM((CHUNK,), jnp.float32),
        pltpu.VMEM_SHARED((N_PAD,), jnp.float32),
        pltpu.SemaphoreType.DMA,
    ],
)
def _sc_hist(dst_hbm, ones_hbm, zeros_hbm, hist_hbm, dst_v, ones_v, hist_sp,
             ssem):
    c = lax.axis_index("c")
    s = lax.axis_index("s")
    wid = c * NS + s

    pltpu.sync_copy(zeros_hbm.at[pl.ds(s * ROWS_PS, ROWS_PS)],
                    hist_sp.at[pl.ds(s * ROWS_PS, ROWS_PS)])
    pltpu.sync_copy(ones_hbm, ones_v)
    pltpu.sync_copy(dst_hbm.at[wid], dst_v)
    plsc.subcore_barrier()

    for j0 in range(8):
        pltpu.async_copy(ones_v, hist_sp.at[dst_v.at[j0]], ssem, add=True)

    def step(j, carry):
        pltpu.make_async_copy(ones_v, hist_sp.at[dst_v.at[j]], ssem).wait()
        pltpu.async_copy(ones_v, hist_sp.at[dst_v.at[j + 8]], ssem, add=True)
        return carry

    lax.fori_loop(0, NG - 8, step, 0)
    for j0 in range(8):
        pltpu.make_async_copy(ones_v, hist_sp.at[dst_v.at[j0]], ssem).wait()

    plsc.subcore_barrier()
    pltpu.sync_copy(hist_sp.at[pl.ds(s * ROWS_PS, ROWS_PS)],
                    hist_hbm.at[pl.ds(c * N_PAD + s * ROWS_PS, ROWS_PS)])


@functools.partial(
    pl.kernel,
    out_type=jax.ShapeDtypeStruct((NC * N_PAD, D_H), jnp.float32),
    mesh=_mesh,
    scratch_types=[
        pltpu.VMEM((NG, CHUNK), jnp.int32),
        pltpu.VMEM((NG, CHUNK), jnp.int32),
        pltpu.VMEM((NB, CHUNK, D_H), jnp.float32),
        pltpu.VMEM_SHARED((N_PAD, D_H), jnp.float32),
        pltpu.SemaphoreType.DMA,
        pltpu.SemaphoreType.DMA,
    ],
    compiler_params=pltpu.CompilerParams(use_tc_tiling_on_sc=False),
)
def _sc_segsum(y_hbm, src_hbm, dst_hbm, zeros_hbm, z_hbm,
               src_v, dst_v, gbuf, z_sp, gsem, ssem):
    c = lax.axis_index("c")
    s = lax.axis_index("s")
    wid = c * NS + s

    pltpu.sync_copy(zeros_hbm.at[pl.ds(s * ROWS_PS, ROWS_PS)],
                    z_sp.at[pl.ds(s * ROWS_PS, ROWS_PS)])
    pltpu.sync_copy(src_hbm.at[wid], src_v)
    pltpu.sync_copy(dst_hbm.at[wid], dst_v)
    plsc.subcore_barrier()

    for b0 in range(PF):
        pltpu.async_copy(y_hbm.at[src_v.at[b0]], gbuf.at[b0], gsem)

    def step(j, carry):
        for b in range(NB):
            g = NB * j + b
            pltpu.make_async_copy(y_hbm.at[src_v.at[g]], gbuf.at[b],
                                  gsem).wait()
            pltpu.async_copy(gbuf.at[b], z_sp.at[dst_v.at[g]], ssem,
                             add=True)
            @pl.when(g >= 2)
            def _():
                pltpu.make_async_copy(gbuf.at[b], z_sp.at[dst_v.at[g]],
                                      ssem).wait()
            @pl.when(g + PF < NG)
            def _():
                pltpu.async_copy(y_hbm.at[src_v.at[g + PF]],
                                 gbuf.at[(b + PF) % NB], gsem)
        return carry

    lax.fori_loop(0, NG // NB, step, 0)
    pltpu.make_async_copy(gbuf.at[0], z_sp.at[dst_v.at[0]], ssem).wait()
    pltpu.make_async_copy(gbuf.at[0], z_sp.at[dst_v.at[0]], ssem).wait()

    plsc.subcore_barrier()
    pltpu.sync_copy(z_sp.at[pl.ds(s * ROWS_PS, ROWS_PS)],
                    z_hbm.at[pl.ds(c * N_PAD + s * ROWS_PS, ROWS_PS)])


def _dinv_col(hist_ref):
    deg = hist_ref[0:1, :] + hist_ref[1:2, :] + 1.0
    return lax.rsqrt(deg).reshape(BLK, 1)


def _tc_scale_matmul_body(hist_ref, x_ref, w_ref, y_ref):
    y_ref[...] = jnp.dot(x_ref[...], w_ref[...],
                         preferred_element_type=jnp.float32) * _dinv_col(hist_ref)


def _tc_mid_body(hist_ref, z0_ref, z1_ref, y_ref, w_ref, b_ref, y2_ref):
    dinv = _dinv_col(hist_ref)
    zsum = z0_ref[...] + z1_ref[...] + y_ref[...]
    h = jnp.maximum(dinv * zsum + b_ref[...], 0.0)
    y2_ref[...] = jnp.dot(h, w_ref[...],
                          preferred_element_type=jnp.float32) * dinv


def _tc_final_body(hist_ref, z0_ref, z1_ref, y_ref, b_ref, out_ref):
    zsum = z0_ref[...] + z1_ref[...] + y_ref[...]
    out_ref[...] = jnp.maximum(_dinv_col(hist_ref) * zsum + b_ref[...], 0.0)


_hist_spec = pl.BlockSpec((2, BLK), lambda i: (0, i))
_row_spec = pl.BlockSpec((BLK, D_H), lambda i: (i, 0))
_z0_spec = pl.BlockSpec((BLK, D_H), lambda i: (i, 0))
_z1_spec = pl.BlockSpec((BLK, D_H), lambda i: (i + GRID, 0))
_b_spec = pl.BlockSpec((1, D_H), lambda i: (0, 0))

_tc_scale_matmul = pl.pallas_call(
    _tc_scale_matmul_body,
    grid=(GRID,),
    in_specs=[_hist_spec,
              pl.BlockSpec((BLK, D_IN), lambda i: (i, 0)),
              pl.BlockSpec((D_IN, D_H), lambda i: (0, 0))],
    out_specs=_row_spec,
    out_shape=jax.ShapeDtypeStruct((N, D_H), jnp.float32),
)

_tc_mid = pl.pallas_call(
    _tc_mid_body,
    grid=(GRID,),
    in_specs=[_hist_spec, _z0_spec, _z1_spec, _row_spec,
              pl.BlockSpec((D_H, D_H), lambda i: (0, 0)), _b_spec],
    out_specs=_row_spec,
    out_shape=jax.ShapeDtypeStruct((N, D_H), jnp.float32),
)

_tc_final = pl.pallas_call(
    _tc_final_body,
    grid=(GRID,),
    in_specs=[_hist_spec, _z0_spec, _z1_spec, _row_spec, _b_spec],
    out_specs=_row_spec,
    out_shape=jax.ShapeDtypeStruct((N, D_H), jnp.float32),
)


def kernel(x, edge_index, W1, b1, W2, b2):
    src = edge_index[0]
    dst = edge_index[1]

    npad_e = E_PAD - E
    pad_src = jnp.arange(npad_e, dtype=jnp.int32) % N
    pad_dst = N + (jnp.arange(npad_e, dtype=jnp.int32) % (N_PAD - N))
    src_p = jnp.concatenate([src, pad_src]).reshape(NW, NG, CHUNK)
    dst_p = jnp.concatenate([dst, pad_dst]).reshape(NW, NG, CHUNK)

    zeros1 = jnp.zeros((N_PAD,), jnp.float32)
    zeros2 = jnp.zeros((N_PAD, D_H), jnp.float32)
    ones_c = jnp.ones((CHUNK,), jnp.float32)

    hist = _sc_hist(dst_p, ones_c, zeros1).reshape(NC, N_PAD)

    y1 = _tc_scale_matmul(hist, x, W1)
    z1 = _sc_segsum(y1, src_p, dst_p, zeros2)
    y2 = _tc_mid(hist, z1, z1, y1, W2, b1.reshape(1, D_H))
    z2 = _sc_segsum(y2, src_p, dst_p, zeros2)
    return _tc_final(hist, z2, z2, y2, b2.reshape(1, D_H))
